# baseline (speedup 1.0000x reference)
import functools

import jax
import jax.numpy as jnp
from jax.experimental import pallas as pl
from jax.experimental.pallas import tpu as pltpu

N = 20000
NPAD = 20480
ROWS = 160
FULL_CHUNKS = N // 128
TAIL = N - FULL_CHUNKS * 128
LANES = 128
NCLS = 90
KDET = 100
NEG = -1e9
SCORE_THRESH = 0.05


def _prep_body(anchors_ref, regression_ref, cls_ref, sp_ref, s0_ref,
               coarse_ref, ct_ref, *, height, width):
    a = anchors_ref[...]
    ya1, xa1, ya2, xa2 = a[0], a[1], a[2], a[3]
    r = regression_ref[0]
    dy, dx, dh, dw = r[0], r[1], r[2], r[3]

    cya = (ya1 + ya2) * 0.5
    cxa = (xa1 + xa2) * 0.5
    ha = ya2 - ya1
    wa = xa2 - xa1
    w = jnp.exp(dw) * wa
    h = jnp.exp(dh) * ha
    yc = dy * ha + cya
    xc = dx * wa + cxa
    bx1 = jnp.clip(xc - w * 0.5, 0.0, width)
    by1 = jnp.clip(yc - h * 0.5, 0.0, height)
    bx2 = jnp.clip(xc + w * 0.5, 0.0, width)
    by2 = jnp.clip(yc + h * 0.5, 0.0, height)

    for g in range(FULL_CHUNKS):
        chunk = cls_ref[0, 128 * g:128 * (g + 1), :]
        ct_ref[g] = jnp.transpose(chunk, (1, 0))
    tail = cls_ref[0, 128 * FULL_CHUNKS:N, :]
    tailp = jnp.concatenate(
        [tail, jnp.full((128 - TAIL, NCLS), -1.0, jnp.float32)], axis=0)
    ct_ref[FULL_CHUNKS] = jnp.transpose(tailp, (1, 0))
    for g in range(FULL_CHUNKS + 1, ROWS):
        ct_ref[g] = jnp.full((NCLS, LANES), -1.0, jnp.float32)

    c3 = ct_ref[...]
    sc = jnp.max(c3, axis=1)
    cit = jax.lax.broadcasted_iota(jnp.int32, (ROWS, NCLS, LANES), 1)
    cls_i = jnp.min(jnp.where(c3 == sc[:, None, :], cit, NCLS), axis=1)
    clsf = cls_i.astype(jnp.float32)

    rowi = jax.lax.broadcasted_iota(jnp.int32, (ROWS, LANES), 0)
    coli = jax.lax.broadcasted_iota(jnp.int32, (ROWS, LANES), 1)
    flat = rowi * LANES + coli
    s0 = jnp.where((flat < N) & (sc > SCORE_THRESH), sc, NEG)

    max_coord = max(height, width) + 1.0
    off = clsf * max_coord
    zz = jnp.zeros((ROWS, LANES), jnp.float32)
    sp_ref[0] = jnp.stack(
        [s0, bx1 + off, by1 + off, bx2 + off, by2 + off, clsf, zz, zz],
        axis=1)
    s0_ref[0] = s0
    coarse_ref[0, 0] = jnp.max(s0, axis=1)


def _nms_loop_body(sp_ref, s0_ref, coarse_ref, out_ref, s_ref, *, batch,
                   height, width):
    max_coord = max(height, width) + 1.0
    riota = jax.lax.broadcasted_iota(jnp.int32, (batch, ROWS), 1)
    bio = jax.lax.broadcasted_iota(jnp.int32, (batch, ROWS), 0)
    lane = jax.lax.broadcasted_iota(jnp.int32, (batch, LANES), 1)
    zb = jnp.zeros((batch, LANES), jnp.float32)

    s_ref[...] = s0_ref[...]
    coarse0 = coarse_ref[...][:, 0, :]

    i0 = jnp.zeros((batch, 1), jnp.int32)
    m0 = jnp.max(coarse0, axis=1, keepdims=True)
    ione = jnp.ones((batch, 1), jnp.int32)
    izero = jnp.zeros((batch, 1), jnp.int32)

    def cond(c):
        i, m_v = c[0], c[2]
        return jnp.any((i < KDET) & (m_v > NEG * 0.5))

    def body(c):
        i, coarse, m_v, sx1, sy1, sx2, sy2, osc, ocl = c
        act = (i < KDET) & (m_v > NEG * 0.5)
        rows_field = jnp.where(coarse == m_v, riota, ROWS)
        rowmin = jnp.min(rows_field, axis=1, keepdims=True)
        rrs = []
        srows = []
        packs = []
        for b in range(batch):
            rr = jnp.min(jnp.where(bio == b, rows_field, ROWS))
            rrs.append(rr)
            srows.append(s_ref[b, pl.ds(rr, 1), :])
            packs.append(sp_ref[b, pl.ds(rr, 1), :, :])
        srow = jnp.concatenate(srows, axis=0)
        pack = jnp.concatenate(packs, axis=0)
        eq = srow == m_v
        li = jnp.min(jnp.where(eq, lane, LANES), axis=1, keepdims=True)
        lm = lane == li
        ext = jnp.sum(jnp.where(lm[:, None, :], pack, 0.0), axis=2)
        xb1 = ext[:, 1:2]
        yb1 = ext[:, 2:3]
        xb2 = ext[:, 3:4]
        yb2 = ext[:, 4:5]
        cb = ext[:, 5:6]
        area_b = (jnp.maximum(xb2 - xb1, 0.0) *
                  jnp.maximum(yb2 - yb1, 0.0))
        iw = jnp.maximum(jnp.minimum(xb2, sx2) - jnp.maximum(xb1, sx1), 0.0)
        ih = jnp.maximum(jnp.minimum(yb2, sy2) - jnp.maximum(yb1, sy1), 0.0)
        inter = iw * ih
        sar = jnp.maximum(sx2 - sx1, 0.0) * jnp.maximum(sy2 - sy1, 0.0)
        denom = sar + area_b - inter + 1e-8
        supp = jnp.any((inter > 0.5 * denom) & (lane < i),
                       axis=1, keepdims=True)
        srow_new = jnp.where(lm & act, NEG, srow)
        for b in range(batch):
            s_ref[b, pl.ds(rrs[b], 1), :] = srow_new[b:b + 1]
        rm = jnp.max(srow_new, axis=1, keepdims=True)
        coarse = jnp.where((riota == rowmin) & act, rm, coarse)
        take = (lane == i) & jnp.logical_not(supp) & act
        sx1 = jnp.where(take, xb1, sx1)
        sy1 = jnp.where(take, yb1, sy1)
        sx2 = jnp.where(take, xb2, sx2)
        sy2 = jnp.where(take, yb2, sy2)
        osc = jnp.where(take, m_v, osc)
        ocl = jnp.where(take, cb, ocl)
        i = i + jnp.where(act & jnp.logical_not(supp), ione, izero)
        m_v = jnp.max(coarse, axis=1, keepdims=True)
        return (i, coarse, m_v, sx1, sy1, sx2, sy2, osc, ocl)

    fin = jax.lax.while_loop(
        cond, body, (i0, coarse0, m0, zb, zb, zb, zb, zb, zb))

    i, _, _, sx1, sy1, sx2, sy2, osc, ocl = fin
    got = lane < i
    offs = ocl * max_coord
    o1 = jnp.where(got, sx1 - offs, 0.0)
    o2 = jnp.where(got, sy1 - offs, 0.0)
    o3 = jnp.where(got, sx2 - offs, 0.0)
    o4 = jnp.where(got, sy2 - offs, 0.0)
    o5 = jnp.where(got, osc, 0.0)
    o6 = jnp.where(got, ocl + 1.0, 0.0)
    out_ref[...] = jnp.stack([o1, o2, o3, o4, o5, o6, zb, zb], axis=1)


def kernel(imgs, anchors, regression, classification):
    height = float(imgs.shape[2])
    width = float(imgs.shape[3])
    B = regression.shape[0]

    at = jnp.transpose(anchors[0], (1, 0))
    at = jnp.pad(at, ((0, 0), (0, NPAD - N))).reshape(4, ROWS, LANES)
    rt = jnp.transpose(regression, (0, 2, 1))
    rt = jnp.pad(rt, ((0, 0), (0, 0), (0, NPAD - N))).reshape(B, 4, ROWS,
                                                              LANES)

    sp, s0, coarse = pl.pallas_call(
        functools.partial(_prep_body, height=height, width=width),
        grid=(B,),
        in_specs=[
            pl.BlockSpec((4, ROWS, LANES), lambda b: (0, 0, 0)),
            pl.BlockSpec((1, 4, ROWS, LANES), lambda b: (b, 0, 0, 0)),
            pl.BlockSpec((1, N, NCLS), lambda b: (b, 0, 0)),
        ],
        out_specs=[
            pl.BlockSpec((1, ROWS, 8, LANES), lambda b: (b, 0, 0, 0)),
            pl.BlockSpec((1, ROWS, LANES), lambda b: (b, 0, 0)),
            pl.BlockSpec((1, 1, ROWS), lambda b: (b, 0, 0)),
        ],
        out_shape=[
            jax.ShapeDtypeStruct((B, ROWS, 8, LANES), jnp.float32),
            jax.ShapeDtypeStruct((B, ROWS, LANES), jnp.float32),
            jax.ShapeDtypeStruct((B, 1, ROWS), jnp.float32),
        ],
        scratch_shapes=[pltpu.VMEM((ROWS, NCLS, LANES), jnp.float32)],
    )(at, rt, classification)

    out_planes = pl.pallas_call(
        functools.partial(_nms_loop_body, batch=B, height=height,
                          width=width),
        out_shape=jax.ShapeDtypeStruct((B, 8, LANES), jnp.float32),
        scratch_shapes=[pltpu.VMEM((B, ROWS, LANES), jnp.float32)],
    )(sp, s0, coarse)

    return jnp.transpose(out_planes[:, :6, :KDET], (0, 2, 1))


# speedup vs baseline: 1.2510x; 1.2510x over previous
import functools

import jax
import jax.numpy as jnp
from jax.experimental import pallas as pl
from jax.experimental.pallas import tpu as pltpu

N = 20000
NPAD = 20480
ROWS = 160
FULL_CHUNKS = N // 128
TAIL = N - FULL_CHUNKS * 128
LANES = 128
NCLS = 90
KDET = 100
NEG = -1e9
SCORE_THRESH = 0.05


def _prep_body(anchors_ref, regression_ref, cls_ref, sp_ref, s0_ref,
               coarse_ref, ct_ref, *, height, width):
    a = anchors_ref[...]
    ya1, xa1, ya2, xa2 = a[0], a[1], a[2], a[3]
    r = regression_ref[0]
    dy, dx, dh, dw = r[0], r[1], r[2], r[3]

    cya = (ya1 + ya2) * 0.5
    cxa = (xa1 + xa2) * 0.5
    ha = ya2 - ya1
    wa = xa2 - xa1
    w = jnp.exp(dw) * wa
    h = jnp.exp(dh) * ha
    yc = dy * ha + cya
    xc = dx * wa + cxa
    bx1 = jnp.clip(xc - w * 0.5, 0.0, width)
    by1 = jnp.clip(yc - h * 0.5, 0.0, height)
    bx2 = jnp.clip(xc + w * 0.5, 0.0, width)
    by2 = jnp.clip(yc + h * 0.5, 0.0, height)

    for g in range(FULL_CHUNKS):
        chunk = cls_ref[0, 128 * g:128 * (g + 1), :]
        ct_ref[g] = jnp.transpose(chunk, (1, 0))
    tail = cls_ref[0, 128 * FULL_CHUNKS:N, :]
    tailp = jnp.concatenate(
        [tail, jnp.full((128 - TAIL, NCLS), -1.0, jnp.float32)], axis=0)
    ct_ref[FULL_CHUNKS] = jnp.transpose(tailp, (1, 0))
    for g in range(FULL_CHUNKS + 1, ROWS):
        ct_ref[g] = jnp.full((NCLS, LANES), -1.0, jnp.float32)

    c3 = ct_ref[...]
    sc = jnp.max(c3, axis=1)
    cit = jax.lax.broadcasted_iota(jnp.int32, (ROWS, NCLS, LANES), 1)
    cls_i = jnp.min(jnp.where(c3 == sc[:, None, :], cit, NCLS), axis=1)
    clsf = cls_i.astype(jnp.float32)

    rowi = jax.lax.broadcasted_iota(jnp.int32, (ROWS, LANES), 0)
    coli = jax.lax.broadcasted_iota(jnp.int32, (ROWS, LANES), 1)
    flat = rowi * LANES + coli
    s0 = jnp.where((flat < N) & (sc > SCORE_THRESH), sc, NEG)

    max_coord = max(height, width) + 1.0
    off = clsf * max_coord
    zz = jnp.zeros((ROWS, LANES), jnp.float32)
    sp_ref[0] = jnp.stack(
        [s0, bx1 + off, by1 + off, bx2 + off, by2 + off, clsf, zz, zz],
        axis=1)
    s0_ref[0] = s0
    coarse_ref[0, 0] = jnp.max(s0, axis=1)


def _nms_loop_body(sp_ref, s0_ref, coarse_ref, out_ref, *refs, batch,
                   height, width):
    s_refs = refs[:batch]
    o_refs = refs[batch:2 * batch]
    max_coord = max(height, width) + 1.0
    riota = jax.lax.broadcasted_iota(jnp.int32, (1, ROWS), 1)
    lane = jax.lax.broadcasted_iota(jnp.int32, (1, LANES), 1)
    zlane = jnp.zeros((1, LANES), jnp.float32)

    coarse0 = []
    for b in range(batch):
        s_refs[b][...] = s0_ref[b]
        coarse0.append(coarse_ref[b, 0:1, :])

    ione = jnp.ones((1, 1), jnp.int32)
    izero = jnp.zeros((1, 1), jnp.int32)

    def bstate(b):
        m0 = jnp.max(coarse0[b], axis=1, keepdims=True)
        return (izero, coarse0[b], m0, zlane, zlane, zlane, zlane)

    def cond(carry):
        alive = [(st[0] < KDET) & (st[2] > NEG * 0.5) for st in carry]
        out = alive[0]
        for x in alive[1:]:
            out = out | x
        return jnp.any(out)

    def body(carry):
        new = []
        for b, st in enumerate(carry):
            i, coarse, m_v, sx1, sy1, sx2, sy2 = st
            act = (i < KDET) & (m_v > NEG * 0.5)
            rr = jnp.min(jnp.where(coarse == m_v, riota, ROWS))
            srow = s_refs[b][pl.ds(rr, 1), :]
            pk = sp_ref[b, pl.ds(rr, 1), :, :]
            eq = srow == m_v
            li = jnp.min(jnp.where(eq, lane, LANES), axis=1, keepdims=True)
            lm = lane == li
            ext = jnp.sum(jnp.where(lm[:, None, :], pk, 0.0),
                          axis=2)
            xb1 = ext[:, 1:2]
            yb1 = ext[:, 2:3]
            xb2 = ext[:, 3:4]
            yb2 = ext[:, 4:5]
            cb = ext[:, 5:6]
            area_b = jnp.maximum(xb2 - xb1, 0.0) * jnp.maximum(yb2 - yb1,
                                                               0.0)
            iw = jnp.maximum(jnp.minimum(xb2, sx2) - jnp.maximum(xb1, sx1),
                             0.0)
            ih = jnp.maximum(jnp.minimum(yb2, sy2) - jnp.maximum(yb1, sy1),
                             0.0)
            inter = iw * ih
            sar = jnp.maximum(sx2 - sx1, 0.0) * jnp.maximum(sy2 - sy1, 0.0)
            denom = sar + area_b - inter + 1e-8
            supp = jnp.any((inter > 0.5 * denom) & (lane < i),
                           axis=1, keepdims=True)
            srow_new = jnp.where(lm & act, NEG, srow)
            s_refs[b][pl.ds(rr, 1), :] = srow_new
            rm = jnp.max(srow_new, axis=1, keepdims=True)
            coarse = jnp.where((riota == rr) & act, rm, coarse)
            take = (lane == i) & jnp.logical_not(supp) & act
            sx1 = jnp.where(take, xb1, sx1)
            sy1 = jnp.where(take, yb1, sy1)
            sx2 = jnp.where(take, xb2, sx2)
            sy2 = jnp.where(take, yb2, sy2)
            o_old = o_refs[b][...]
            vals = jnp.concatenate(
                [jnp.broadcast_to(m_v, (1, LANES)),
                 jnp.broadcast_to(cb, (1, LANES))], axis=0)
            o_refs[b][...] = jnp.where(take, vals, o_old)
            i = i + jnp.where(act & jnp.logical_not(supp), ione, izero)
            m_v = jnp.max(coarse, axis=1, keepdims=True)
            new.append((i, coarse, m_v, sx1, sy1, sx2, sy2))
        return tuple(new)

    fin = jax.lax.while_loop(cond, body, tuple(bstate(b)
                                               for b in range(batch)))

    for b in range(batch):
        i, _, _, sx1, sy1, sx2, sy2 = fin[b]
        ssc = o_refs[b][0:1, :]
        scl = o_refs[b][1:2, :]
        got = lane < i
        offs = scl * max_coord
        o1 = jnp.where(got, sx1 - offs, 0.0)
        o2 = jnp.where(got, sy1 - offs, 0.0)
        o3 = jnp.where(got, sx2 - offs, 0.0)
        o4 = jnp.where(got, sy2 - offs, 0.0)
        o5 = jnp.where(got, ssc, 0.0)
        o6 = jnp.where(got, scl + 1.0, 0.0)
        out_ref[b] = jnp.concatenate([o1, o2, o3, o4, o5, o6, zlane, zlane],
                                     axis=0)


def kernel(imgs, anchors, regression, classification):
    height = float(imgs.shape[2])
    width = float(imgs.shape[3])
    B = regression.shape[0]

    at = jnp.transpose(anchors[0], (1, 0))
    at = jnp.pad(at, ((0, 0), (0, NPAD - N))).reshape(4, ROWS, LANES)
    rt = jnp.transpose(regression, (0, 2, 1))
    rt = jnp.pad(rt, ((0, 0), (0, 0), (0, NPAD - N))).reshape(B, 4, ROWS,
                                                              LANES)

    sp, s0, coarse = pl.pallas_call(
        functools.partial(_prep_body, height=height, width=width),
        grid=(B,),
        in_specs=[
            pl.BlockSpec((4, ROWS, LANES), lambda b: (0, 0, 0)),
            pl.BlockSpec((1, 4, ROWS, LANES), lambda b: (b, 0, 0, 0)),
            pl.BlockSpec((1, N, NCLS), lambda b: (b, 0, 0)),
        ],
        out_specs=[
            pl.BlockSpec((1, ROWS, 8, LANES), lambda b: (b, 0, 0, 0)),
            pl.BlockSpec((1, ROWS, LANES), lambda b: (b, 0, 0)),
            pl.BlockSpec((1, 1, ROWS), lambda b: (b, 0, 0)),
        ],
        out_shape=[
            jax.ShapeDtypeStruct((B, ROWS, 8, LANES), jnp.float32),
            jax.ShapeDtypeStruct((B, ROWS, LANES), jnp.float32),
            jax.ShapeDtypeStruct((B, 1, ROWS), jnp.float32),
        ],
        scratch_shapes=[pltpu.VMEM((ROWS, NCLS, LANES), jnp.float32)],
    )(at, rt, classification)

    out_planes = pl.pallas_call(
        functools.partial(_nms_loop_body, batch=B, height=height,
                          width=width),
        out_shape=jax.ShapeDtypeStruct((B, 8, LANES), jnp.float32),
        scratch_shapes=([pltpu.VMEM((ROWS, LANES), jnp.float32)
                         for _ in range(B)] +
                        [pltpu.VMEM((2, LANES), jnp.float32)
                         for _ in range(B)]),
    )(sp, s0, coarse)

    return jnp.transpose(out_planes[:, :6, :KDET], (0, 2, 1))


# speedup vs baseline: 1.5935x; 1.2738x over previous
import functools

import jax
import jax.numpy as jnp
from jax.experimental import pallas as pl
from jax.experimental.pallas import tpu as pltpu

N = 20000
NPAD = 20480
ROWS = 160
FULL_CHUNKS = N // 128
TAIL = N - FULL_CHUNKS * 128
LANES = 128
NCLS = 90
KDET = 100
NEG = -1e9
SCORE_THRESH = 0.05


def _prep_body(anchors_ref, regression_ref, cls_ref, sp_ref, coarse_ref,
               ct_ref, *, height, width):
    a = anchors_ref[...]
    ya1, xa1, ya2, xa2 = a[0], a[1], a[2], a[3]
    r = regression_ref[0]
    dy, dx, dh, dw = r[0], r[1], r[2], r[3]

    cya = (ya1 + ya2) * 0.5
    cxa = (xa1 + xa2) * 0.5
    ha = ya2 - ya1
    wa = xa2 - xa1
    w = jnp.exp(dw) * wa
    h = jnp.exp(dh) * ha
    yc = dy * ha + cya
    xc = dx * wa + cxa
    bx1 = jnp.clip(xc - w * 0.5, 0.0, width)
    by1 = jnp.clip(yc - h * 0.5, 0.0, height)
    bx2 = jnp.clip(xc + w * 0.5, 0.0, width)
    by2 = jnp.clip(yc + h * 0.5, 0.0, height)

    for g in range(FULL_CHUNKS):
        chunk = cls_ref[0, 128 * g:128 * (g + 1), :]
        ct_ref[g] = jnp.transpose(chunk, (1, 0))
    tail = cls_ref[0, 128 * FULL_CHUNKS:N, :]
    tailp = jnp.concatenate(
        [tail, jnp.full((128 - TAIL, NCLS), -1.0, jnp.float32)], axis=0)
    ct_ref[FULL_CHUNKS] = jnp.transpose(tailp, (1, 0))
    for g in range(FULL_CHUNKS + 1, ROWS):
        ct_ref[g] = jnp.full((NCLS, LANES), -1.0, jnp.float32)

    c3 = ct_ref[...]
    sc = jnp.max(c3, axis=1)
    cit = jax.lax.broadcasted_iota(jnp.int32, (ROWS, NCLS, LANES), 1)
    cls_i = jnp.min(jnp.where(c3 == sc[:, None, :], cit, NCLS), axis=1)
    clsf = cls_i.astype(jnp.float32)

    rowi = jax.lax.broadcasted_iota(jnp.int32, (ROWS, LANES), 0)
    coli = jax.lax.broadcasted_iota(jnp.int32, (ROWS, LANES), 1)
    flat = rowi * LANES + coli
    s0 = jnp.where((flat < N) & (sc > SCORE_THRESH), sc, NEG)

    max_coord = max(height, width) + 1.0
    off = clsf * max_coord
    sp_ref[0, 0] = s0
    sp_ref[0, 1] = bx1 + off
    sp_ref[0, 2] = by1 + off
    sp_ref[0, 3] = bx2 + off
    sp_ref[0, 4] = by2 + off
    sp_ref[0, 5] = clsf
    coarse_ref[0, 0] = jnp.max(s0, axis=1)


def _nms_loop_body(sp_ref, coarse_ref, out_ref, *refs, batch, height, width):
    s_refs = refs[:batch]
    o_refs = refs[batch:2 * batch]
    max_coord = max(height, width) + 1.0
    riota = jax.lax.broadcasted_iota(jnp.int32, (1, ROWS), 1)
    lane = jax.lax.broadcasted_iota(jnp.int32, (1, LANES), 1)
    zlane = jnp.zeros((1, LANES), jnp.float32)

    coarse0 = []
    for b in range(batch):
        s_refs[b][...] = sp_ref[b, 0]
        coarse0.append(coarse_ref[b, 0:1, :])

    ione = jnp.ones((1, 1), jnp.int32)
    izero = jnp.zeros((1, 1), jnp.int32)

    def bstate(b):
        co = coarse0[b]
        m0 = jnp.max(co, axis=1, keepdims=True)
        rowf = jnp.where(co == m0, riota, ROWS)
        rrv0 = jnp.min(rowf, axis=1, keepdims=True)
        rrs0 = jnp.min(rowf)
        srow0 = s_refs[b][pl.ds(rrs0, 1), :]
        return (izero, co, m0, rrs0, rrv0, srow0,
                zlane, zlane, zlane, zlane)

    def cond(carry):
        alive = [(st[0] < KDET) & (st[2] > NEG * 0.5) for st in carry]
        out = alive[0]
        for x in alive[1:]:
            out = out | x
        return jnp.any(out)

    def body(carry):
        new = []
        for b, st in enumerate(carry):
            i, coarse, m_v, rrs, rrv, srow, sx1, sy1, sx2, sy2 = st
            act = (i < KDET) & (m_v > NEG * 0.5)
            prow = sp_ref[b, 1:6, pl.ds(rrs, 1), :]
            eq = srow == m_v
            li = jnp.min(jnp.where(eq, lane, LANES), axis=1, keepdims=True)
            lm = lane == li
            ext = jnp.sum(jnp.where(lm[None], prow, 0.0),
                          axis=2, keepdims=True)
            xb1 = ext[0]
            yb1 = ext[1]
            xb2 = ext[2]
            yb2 = ext[3]
            cb = ext[4]
            area_b = jnp.maximum(xb2 - xb1, 0.0) * jnp.maximum(yb2 - yb1,
                                                               0.0)
            iw = jnp.maximum(jnp.minimum(xb2, sx2) - jnp.maximum(xb1, sx1),
                             0.0)
            ih = jnp.maximum(jnp.minimum(yb2, sy2) - jnp.maximum(yb1, sy1),
                             0.0)
            inter = iw * ih
            sar = jnp.maximum(sx2 - sx1, 0.0) * jnp.maximum(sy2 - sy1, 0.0)
            denom = sar + area_b - inter + 1e-8
            supp = jnp.any((inter > 0.5 * denom) & (lane < i),
                           axis=1, keepdims=True)
            srow_new = jnp.where(lm & act, NEG, srow)
            s_refs[b][pl.ds(rrs, 1), :] = srow_new
            rm = jnp.max(srow_new, axis=1, keepdims=True)
            aco = jnp.where(riota == rrv, NEG, coarse)
            altm = jnp.max(aco, axis=1, keepdims=True)
            rowf = jnp.where(aco == altm, riota, ROWS)
            altv = jnp.min(rowf, axis=1, keepdims=True)
            alts = jnp.min(rowf)
            salt = s_refs[b][pl.ds(alts, 1), :]
            coarse = jnp.where((riota == rrv) & act, rm, coarse)
            use_cur = (rm > altm) | ((rm == altm) & (rrv < altv))
            m_new = jnp.maximum(rm, altm)
            srow_nx = jnp.where(use_cur, srow_new, salt)
            rrv_nx = jnp.where(use_cur, rrv, altv)
            rrs_nx = jnp.where(jnp.any(use_cur), rrs, alts)
            m_v = jnp.where(act, m_new, m_v)
            srow = jnp.where(act, srow_nx, srow)
            rrv = jnp.where(act, rrv_nx, rrv)
            rrs = jnp.where(jnp.any(act), rrs_nx, rrs)
            take = (lane == i) & jnp.logical_not(supp) & act
            sx1 = jnp.where(take, xb1, sx1)
            sy1 = jnp.where(take, yb1, sy1)
            sx2 = jnp.where(take, xb2, sx2)
            sy2 = jnp.where(take, yb2, sy2)
            o_old = o_refs[b][...]
            vals = jnp.concatenate(
                [jnp.broadcast_to(st[2], (1, LANES)),
                 jnp.broadcast_to(cb, (1, LANES))], axis=0)
            o_refs[b][...] = jnp.where(take, vals, o_old)
            i = i + jnp.where(act & jnp.logical_not(supp), ione, izero)
            new.append((i, coarse, m_v, rrs, rrv, srow, sx1, sy1, sx2, sy2))
        return tuple(new)

    fin = jax.lax.while_loop(cond, body, tuple(bstate(b)
                                               for b in range(batch)))

    for b in range(batch):
        i = fin[b][0]
        sx1, sy1, sx2, sy2 = fin[b][6:10]
        ssc = o_refs[b][0:1, :]
        scl = o_refs[b][1:2, :]
        got = lane < i
        offs = scl * max_coord
        o1 = jnp.where(got, sx1 - offs, 0.0)
        o2 = jnp.where(got, sy1 - offs, 0.0)
        o3 = jnp.where(got, sx2 - offs, 0.0)
        o4 = jnp.where(got, sy2 - offs, 0.0)
        o5 = jnp.where(got, ssc, 0.0)
        o6 = jnp.where(got, scl + 1.0, 0.0)
        out_ref[b] = jnp.concatenate([o1, o2, o3, o4, o5, o6, zlane, zlane],
                                     axis=0)


def kernel(imgs, anchors, regression, classification):
    height = float(imgs.shape[2])
    width = float(imgs.shape[3])
    B = regression.shape[0]

    at = jnp.transpose(anchors[0], (1, 0))
    at = jnp.pad(at, ((0, 0), (0, NPAD - N))).reshape(4, ROWS, LANES)
    rt = jnp.transpose(regression, (0, 2, 1))
    rt = jnp.pad(rt, ((0, 0), (0, 0), (0, NPAD - N))).reshape(B, 4, ROWS,
                                                              LANES)

    sp, coarse = pl.pallas_call(
        functools.partial(_prep_body, height=height, width=width),
        grid=(B,),
        in_specs=[
            pl.BlockSpec((4, ROWS, LANES), lambda b: (0, 0, 0)),
            pl.BlockSpec((1, 4, ROWS, LANES), lambda b: (b, 0, 0, 0)),
            pl.BlockSpec((1, N, NCLS), lambda b: (b, 0, 0)),
        ],
        out_specs=[
            pl.BlockSpec((1, 6, ROWS, LANES), lambda b: (b, 0, 0, 0)),
            pl.BlockSpec((1, 1, ROWS), lambda b: (b, 0, 0)),
        ],
        out_shape=[
            jax.ShapeDtypeStruct((B, 6, ROWS, LANES), jnp.float32),
            jax.ShapeDtypeStruct((B, 1, ROWS), jnp.float32),
        ],
        scratch_shapes=[pltpu.VMEM((ROWS, NCLS, LANES), jnp.float32)],
    )(at, rt, classification)

    out_planes = pl.pallas_call(
        functools.partial(_nms_loop_body, batch=B, height=height,
                          width=width),
        out_shape=jax.ShapeDtypeStruct((B, 8, LANES), jnp.float32),
        scratch_shapes=([pltpu.VMEM((ROWS, LANES), jnp.float32)
                         for _ in range(B)] +
                        [pltpu.VMEM((2, LANES), jnp.float32)
                         for _ in range(B)]),
    )(sp, coarse)

    return jnp.transpose(out_planes[:, :6, :KDET], (0, 2, 1))


# speedup vs baseline: 1.6010x; 1.0047x over previous
import functools

import jax
import jax.numpy as jnp
from jax.experimental import pallas as pl
from jax.experimental.pallas import tpu as pltpu

N = 20000
NPAD = 20480
ROWS = 160
FULL_CHUNKS = N // 128
TAIL = N - FULL_CHUNKS * 128
LANES = 128
NCLS = 90
KDET = 100
NEG = -1e9
SCORE_THRESH = 0.05


def _prep_body(anchors_ref, regression_ref, cls_ref, sp_ref, coarse_ref,
               ct_ref, *, height, width):
    a = anchors_ref[...]
    ya1, xa1, ya2, xa2 = a[0], a[1], a[2], a[3]
    r = regression_ref[0]
    dy, dx, dh, dw = r[0], r[1], r[2], r[3]

    cya = (ya1 + ya2) * 0.5
    cxa = (xa1 + xa2) * 0.5
    ha = ya2 - ya1
    wa = xa2 - xa1
    w = jnp.exp(dw) * wa
    h = jnp.exp(dh) * ha
    yc = dy * ha + cya
    xc = dx * wa + cxa
    bx1 = jnp.clip(xc - w * 0.5, 0.0, width)
    by1 = jnp.clip(yc - h * 0.5, 0.0, height)
    bx2 = jnp.clip(xc + w * 0.5, 0.0, width)
    by2 = jnp.clip(yc + h * 0.5, 0.0, height)

    for g in range(FULL_CHUNKS):
        chunk = cls_ref[0, 128 * g:128 * (g + 1), :]
        ct_ref[g] = jnp.transpose(chunk, (1, 0))
    tail = cls_ref[0, 128 * FULL_CHUNKS:N, :]
    tailp = jnp.concatenate(
        [tail, jnp.full((128 - TAIL, NCLS), -1.0, jnp.float32)], axis=0)
    ct_ref[FULL_CHUNKS] = jnp.transpose(tailp, (1, 0))
    for g in range(FULL_CHUNKS + 1, ROWS):
        ct_ref[g] = jnp.full((NCLS, LANES), -1.0, jnp.float32)

    c3 = ct_ref[...]
    sc = jnp.max(c3, axis=1)
    cit = jax.lax.broadcasted_iota(jnp.int32, (ROWS, NCLS, LANES), 1)
    cls_i = jnp.min(jnp.where(c3 == sc[:, None, :], cit, NCLS), axis=1)
    clsf = cls_i.astype(jnp.float32)

    rowi = jax.lax.broadcasted_iota(jnp.int32, (ROWS, LANES), 0)
    coli = jax.lax.broadcasted_iota(jnp.int32, (ROWS, LANES), 1)
    flat = rowi * LANES + coli
    s0 = jnp.where((flat < N) & (sc > SCORE_THRESH), sc, NEG)

    max_coord = max(height, width) + 1.0
    off = clsf * max_coord
    sp_ref[0, 0] = s0
    sp_ref[0, 1] = bx1 + off
    sp_ref[0, 2] = by1 + off
    sp_ref[0, 3] = bx2 + off
    sp_ref[0, 4] = by2 + off
    sp_ref[0, 5] = clsf
    coarse_ref[0, 0] = jnp.max(s0, axis=1)


def _nms_loop_body(sp_ref, coarse_ref, out_ref, *refs, batch, height, width):
    s_refs = refs[:batch]
    max_coord = max(height, width) + 1.0
    riota = jax.lax.broadcasted_iota(jnp.int32, (1, ROWS), 1)
    lane = jax.lax.broadcasted_iota(jnp.int32, (1, LANES), 1)
    zlane = jnp.zeros((1, LANES), jnp.float32)

    coarse0 = []
    for b in range(batch):
        s_refs[b][...] = sp_ref[b, 0]
        coarse0.append(coarse_ref[b, 0:1, :])

    ione = jnp.ones((1, 1), jnp.int32)
    izero = jnp.zeros((1, 1), jnp.int32)

    def bstate(b):
        co = coarse0[b]
        m0 = jnp.max(co, axis=1, keepdims=True)
        rowf = jnp.where(co == m0, riota, ROWS)
        rrv0 = jnp.min(rowf, axis=1, keepdims=True)
        rrs0 = jnp.min(rowf)
        srow0 = s_refs[b][pl.ds(rrs0, 1), :]
        return (izero, co, m0, rrs0, rrv0, srow0,
                zlane, zlane, zlane, zlane, zlane, zlane)

    def cond(carry):
        alive = [(st[0] < KDET) & (st[2] > NEG * 0.5) for st in carry]
        out = alive[0]
        for x in alive[1:]:
            out = out | x
        return jnp.any(out)

    def body(carry):
        new = []
        for b, st in enumerate(carry):
            (i, coarse, m_v, rrs, rrv, srow,
             sx1, sy1, sx2, sy2, osc, ocl) = st
            act = (i < KDET) & (m_v > NEG * 0.5)
            prow = sp_ref[b, 1:6, pl.ds(rrs, 1), :]
            eq = srow == m_v
            li = jnp.min(jnp.where(eq, lane, LANES), axis=1, keepdims=True)
            lm = lane == li
            ext = jnp.sum(jnp.where(lm[None], prow, 0.0),
                          axis=2, keepdims=True)
            xb1 = ext[0]
            yb1 = ext[1]
            xb2 = ext[2]
            yb2 = ext[3]
            cb = ext[4]
            area_b = jnp.maximum(xb2 - xb1, 0.0) * jnp.maximum(yb2 - yb1,
                                                               0.0)
            iw = jnp.maximum(jnp.minimum(xb2, sx2) - jnp.maximum(xb1, sx1),
                             0.0)
            ih = jnp.maximum(jnp.minimum(yb2, sy2) - jnp.maximum(yb1, sy1),
                             0.0)
            inter = iw * ih
            sar = jnp.maximum(sx2 - sx1, 0.0) * jnp.maximum(sy2 - sy1, 0.0)
            denom = sar + area_b - inter + 1e-8
            supp = jnp.any((inter > 0.5 * denom) & (lane < i),
                           axis=1, keepdims=True)
            srow_new = jnp.where(lm & act, NEG, srow)
            s_refs[b][pl.ds(rrs, 1), :] = srow_new
            rm = jnp.max(srow_new, axis=1, keepdims=True)
            aco = jnp.where(riota == rrv, NEG, coarse)
            altm = jnp.max(aco, axis=1, keepdims=True)
            rowf = jnp.where(aco == altm, riota, ROWS)
            altv = jnp.min(rowf, axis=1, keepdims=True)
            alts = jnp.min(rowf)
            salt = s_refs[b][pl.ds(alts, 1), :]
            coarse = jnp.where((riota == rrv) & act, rm, coarse)
            use_cur = (rm > altm) | ((rm == altm) & (rrv < altv))
            m_new = jnp.maximum(rm, altm)
            srow_nx = jnp.where(use_cur, srow_new, salt)
            rrv_nx = jnp.where(use_cur, rrv, altv)
            rrs_nx = jnp.where(jnp.any(use_cur), rrs, alts)
            m_v = jnp.where(act, m_new, m_v)
            srow = jnp.where(act, srow_nx, srow)
            rrv = jnp.where(act, rrv_nx, rrv)
            rrs = jnp.where(jnp.any(act), rrs_nx, rrs)
            take = (lane == i) & jnp.logical_not(supp) & act
            sx1 = jnp.where(take, xb1, sx1)
            sy1 = jnp.where(take, yb1, sy1)
            sx2 = jnp.where(take, xb2, sx2)
            sy2 = jnp.where(take, yb2, sy2)
            osc = jnp.where(take, st[2], osc)
            ocl = jnp.where(take, cb, ocl)
            i = i + jnp.where(act & jnp.logical_not(supp), ione, izero)
            new.append((i, coarse, m_v, rrs, rrv, srow,
                        sx1, sy1, sx2, sy2, osc, ocl))
        return tuple(new)

    fin = jax.lax.while_loop(cond, body, tuple(bstate(b)
                                               for b in range(batch)))

    for b in range(batch):
        i = fin[b][0]
        sx1, sy1, sx2, sy2, ssc, scl = fin[b][6:12]
        got = lane < i
        offs = scl * max_coord
        o1 = jnp.where(got, sx1 - offs, 0.0)
        o2 = jnp.where(got, sy1 - offs, 0.0)
        o3 = jnp.where(got, sx2 - offs, 0.0)
        o4 = jnp.where(got, sy2 - offs, 0.0)
        o5 = jnp.where(got, ssc, 0.0)
        o6 = jnp.where(got, scl + 1.0, 0.0)
        out_ref[b] = jnp.concatenate([o1, o2, o3, o4, o5, o6, zlane, zlane],
                                     axis=0)


def kernel(imgs, anchors, regression, classification):
    height = float(imgs.shape[2])
    width = float(imgs.shape[3])
    B = regression.shape[0]

    at = jnp.transpose(anchors[0], (1, 0))
    at = jnp.pad(at, ((0, 0), (0, NPAD - N))).reshape(4, ROWS, LANES)
    rt = jnp.transpose(regression, (0, 2, 1))
    rt = jnp.pad(rt, ((0, 0), (0, 0), (0, NPAD - N))).reshape(B, 4, ROWS,
                                                              LANES)

    sp, coarse = pl.pallas_call(
        functools.partial(_prep_body, height=height, width=width),
        grid=(B,),
        in_specs=[
            pl.BlockSpec((4, ROWS, LANES), lambda b: (0, 0, 0)),
            pl.BlockSpec((1, 4, ROWS, LANES), lambda b: (b, 0, 0, 0)),
            pl.BlockSpec((1, N, NCLS), lambda b: (b, 0, 0)),
        ],
        out_specs=[
            pl.BlockSpec((1, 6, ROWS, LANES), lambda b: (b, 0, 0, 0)),
            pl.BlockSpec((1, 1, ROWS), lambda b: (b, 0, 0)),
        ],
        out_shape=[
            jax.ShapeDtypeStruct((B, 6, ROWS, LANES), jnp.float32),
            jax.ShapeDtypeStruct((B, 1, ROWS), jnp.float32),
        ],
        scratch_shapes=[pltpu.VMEM((ROWS, NCLS, LANES), jnp.float32)],
    )(at, rt, classification)

    out_planes = pl.pallas_call(
        functools.partial(_nms_loop_body, batch=B, height=height,
                          width=width),
        out_shape=jax.ShapeDtypeStruct((B, 8, LANES), jnp.float32),
        scratch_shapes=[pltpu.VMEM((ROWS, LANES), jnp.float32)
                        for _ in range(B)],
    )(sp, coarse)

    return jnp.transpose(out_planes[:, :6, :KDET], (0, 2, 1))
